# Initial kernel scaffold; baseline (speedup 1.0000x reference)
import jax
import jax.numpy as jnp
from jax.experimental import pallas as pl


def kernel(x):
    raise NotImplementedError("write your pallas kernel here")


# speedup vs baseline: 442.3728x; 442.3728x over previous
import jax
import jax.numpy as jnp
from jax.experimental import pallas as pl
from jax.experimental.pallas import tpu as pltpu

N_RAW = 5000
N = 5120
B = 512
NB = N // B
CONF = 0.25
IOU = 0.45
F32 = jnp.float32


def _iotai(shape, dim):
    return jax.lax.broadcasted_iota(jnp.int32, shape, dim)


def _iota(shape, dim):
    return _iotai(shape, dim).astype(F32)


_EYE = None


def _eye():
    return (_iotai((B, B), 0) == _iotai((B, B), 1))


def _row_to_col(v):
    return jnp.sum(jnp.where(_eye(), v, 0.0), axis=1, keepdims=True)


def _col_to_row(v):
    return jnp.sum(jnp.where(_eye(), v, 0.0), axis=0, keepdims=True)


def _iou_mask(bx1, by1, bx2, by2, tx1, ty1, tx2, ty2):
    ix1 = jnp.maximum(bx1, tx1)
    iy1 = jnp.maximum(by1, ty1)
    ix2 = jnp.minimum(bx2, tx2)
    iy2 = jnp.minimum(by2, ty2)
    inter = jnp.clip(ix2 - ix1, 0.0) * jnp.clip(iy2 - iy1, 0.0)
    a1 = jnp.clip(bx2 - bx1, 0.0) * jnp.clip(by2 - by1, 0.0)
    a2 = jnp.clip(tx2 - tx1, 0.0) * jnp.clip(ty2 - ty1, 0.0)
    iou = inter / (a1 + a2 - inter + 1e-9)
    return (iou > IOU).astype(F32)


def _nms_kernel(xt_ref, out_ref, dt_ref, st_ref, rankr_ref, krow_ref,
                m_ref):
    f32 = F32
    xr = xt_ref[...]
    cxr = xr[0:1, :] * 640.0
    cyr = xr[1:2, :] * 640.0
    wr = xr[2:3, :] * 640.0
    hr = xr[3:4, :] * 640.0
    sr = xr[4:5, :] * xr[5:6, :]
    smr = jnp.where(sr > CONF, sr, -1.0)
    dt_ref[...] = jnp.concatenate(
        [cxr - wr / 2, cyr - hr / 2, cxr + wr / 2, cyr + hr / 2, smr,
         jnp.zeros((3, N), f32)], axis=0)

    sjc = []
    for j in range(NB):
        sjc.append(_row_to_col(dt_ref[4:5, j * B:(j + 1) * B]))
    for t in range(NB):
        si = dt_ref[4:5, t * B:(t + 1) * B]
        ii = _iotai((1, B), 1) + t * B
        acc = jnp.zeros((1, B), f32)
        for j in range(NB):
            jj = _iotai((B, 1), 0) + j * B
            hit = (sjc[j] > si) | ((sjc[j] == si) & (jj < ii))
            acc = acc + jnp.sum(hit.astype(f32), axis=0, keepdims=True)
        rankr_ref[0:1, t * B:(t + 1) * B] = acc

    dn = (((1,), (0,)), ((), ()))
    rkc = []
    for j in range(NB):
        rkc.append(_row_to_col(rankr_ref[0:1, j * B:(j + 1) * B]))
    for t in range(NB):
        kkr = _iota((1, B), 1) + t * B
        acc = jnp.zeros((8, B), f32)
        for j in range(NB):
            q = (rkc[j] == kkr).astype(f32)
            acc = acc + jax.lax.dot_general(
                dt_ref[:, j * B:(j + 1) * B], q, dn,
                preferred_element_type=f32,
                precision=jax.lax.Precision.HIGHEST)
        st_ref[:, t * B:(t + 1) * B] = acc

    krow_ref[...] = (st_ref[4:5, :] > CONF).astype(f32)

    for b in range(NB):
        bs = slice(b * B, (b + 1) * B)
        v_row = krow_ref[0:1, bs]

        @pl.when(jnp.sum(v_row) > 0.0)
        def _process(b=b, bs=bs, v_row=v_row):
            tx1 = st_ref[0:1, bs]
            ty1 = st_ref[1:2, bs]
            tx2 = st_ref[2:3, bs]
            ty2 = st_ref[3:4, bs]
            bx1 = _row_to_col(tx1)
            by1 = _row_to_col(ty1)
            bx2 = _row_to_col(tx2)
            by2 = _row_to_col(ty2)
            m_ref[...] = _iou_mask(bx1, by1, bx2, by2, tx1, ty1, tx2, ty2)
            v_col = _row_to_col(v_row)

            ri = _iotai((B, B), 0)
            ci = _iotai((B, B), 1)
            up = (ri < ci).astype(F32)
            lo = (ri > ci).astype(F32)

            def cond(st):
                _, t, diff = st
                return (diff > 0.0) & (t < B + 2)

            def body(st):
                kc, t, _ = st
                m = m_ref[...]
                sup_r = jnp.max(m * up * kc, axis=0, keepdims=True)
                kr = v_row * (1.0 - sup_r)
                sup_c = jnp.max(m * lo * kr, axis=1, keepdims=True)
                kc2 = v_col * (1.0 - sup_c)
                return kc2, t + 1, jnp.sum(jnp.abs(kc2 - kc))

            kc_fin, _, _ = jax.lax.while_loop(
                cond, body, (v_col, jnp.int32(0), jnp.float32(1.0)))
            krow_ref[0:1, bs] = _col_to_row(kc_fin)

            @pl.when(jnp.sum(kc_fin) > 0.0)
            def _cross():
                for c in range(b + 1, NB):
                    cs = slice(c * B, (c + 1) * B)
                    ka = krow_ref[0:1, cs]

                    @pl.when(jnp.sum(ka) > 0.0)
                    def _one(c=c, cs=cs, ka=ka):
                        mt = _iou_mask(bx1, by1, bx2, by2,
                                       st_ref[0:1, cs], st_ref[1:2, cs],
                                       st_ref[2:3, cs], st_ref[3:4, cs])
                        sup = jnp.max(mt * kc_fin, axis=0, keepdims=True)
                        krow_ref[0:1, cs] = ka * (1.0 - sup)

    out_ref[...] = st_ref[...] * krow_ref[...]


@jax.jit
def kernel(x):
    p = x[0]
    xt = jnp.zeros((8, N), F32).at[:6, :N_RAW].set(p.T)
    out_t = pl.pallas_call(
        _nms_kernel,
        out_shape=jax.ShapeDtypeStruct((8, N), F32),
        scratch_shapes=[
            pltpu.VMEM((8, N), F32),
            pltpu.VMEM((8, N), F32),
            pltpu.VMEM((1, N), F32),
            pltpu.VMEM((1, N), F32),
            pltpu.VMEM((B, B), F32),
        ],
    )(xt)
    return out_t[:5, :N_RAW].T


# speedup vs baseline: 504.7016x; 1.1409x over previous
import jax
import jax.numpy as jnp
from jax.experimental import pallas as pl
from jax.experimental.pallas import tpu as pltpu

N_RAW = 5000
N = 5120
B = 512
NB = N // B
CONF = 0.25
IOU = 0.45
F32 = jnp.float32


def _iotai(shape, dim):
    return jax.lax.broadcasted_iota(jnp.int32, shape, dim)


def _iota(shape, dim):
    return _iotai(shape, dim).astype(F32)


_EYE = None


def _eye():
    return (_iotai((B, B), 0) == _iotai((B, B), 1))


def _row_to_col(v):
    return jnp.sum(jnp.where(_eye(), v, 0.0), axis=1, keepdims=True)


def _col_to_row(v):
    return jnp.sum(jnp.where(_eye(), v, 0.0), axis=0, keepdims=True)


def _iou_mask(bx1, by1, bx2, by2, tx1, ty1, tx2, ty2):
    ix1 = jnp.maximum(bx1, tx1)
    iy1 = jnp.maximum(by1, ty1)
    ix2 = jnp.minimum(bx2, tx2)
    iy2 = jnp.minimum(by2, ty2)
    inter = jnp.clip(ix2 - ix1, 0.0) * jnp.clip(iy2 - iy1, 0.0)
    a1 = jnp.clip(bx2 - bx1, 0.0) * jnp.clip(by2 - by1, 0.0)
    a2 = jnp.clip(tx2 - tx1, 0.0) * jnp.clip(ty2 - ty1, 0.0)
    iou = inter / (a1 + a2 - inter + 1e-9)
    return (iou > IOU).astype(F32)


def _nms_kernel(xt_ref, out_ref, dt_ref, st_ref, rankr_ref, krow_ref,
                m_ref):
    f32 = F32
    xr = xt_ref[...]
    cxr = xr[0:1, :] * 640.0
    cyr = xr[1:2, :] * 640.0
    wr = xr[2:3, :] * 640.0
    hr = xr[3:4, :] * 640.0
    sr = xr[4:5, :] * xr[5:6, :]
    smr = jnp.where(sr > CONF, sr, -1.0)
    dt_ref[...] = jnp.concatenate(
        [cxr - wr / 2, cyr - hr / 2, cxr + wr / 2, cyr + hr / 2, smr,
         jnp.zeros((3, N), f32)], axis=0)

    sjc = []
    for j in range(NB):
        sjc.append(_row_to_col(dt_ref[4:5, j * B:(j + 1) * B]))
    tri = _iotai((B, 1), 0) < _iotai((1, B), 1)
    for t in range(NB):
        si = dt_ref[4:5, t * B:(t + 1) * B]
        acc = jnp.zeros((1, B), f32)
        for j in range(NB):
            if j < t:
                hit = sjc[j] >= si
            elif j > t:
                hit = sjc[j] > si
            else:
                hit = (sjc[j] > si) | ((sjc[j] == si) & tri)
            acc = acc + jnp.sum(hit.astype(f32), axis=0, keepdims=True)
        rankr_ref[0:1, t * B:(t + 1) * B] = acc

    dn = (((1,), (0,)), ((), ()))
    rkc = []
    for j in range(NB):
        rkc.append(_row_to_col(rankr_ref[0:1, j * B:(j + 1) * B]))
    nvalid = jnp.sum((smr > CONF).astype(f32))
    for t in range(NB):
        ts = slice(t * B, (t + 1) * B)

        @pl.when(nvalid > float(t * B))
        def _gather(ts=ts, t=t):
            kkr = _iota((1, B), 1) + t * B
            acc = jnp.zeros((8, B), f32)
            for j in range(NB):
                q = (rkc[j] == kkr).astype(f32)
                acc = acc + jax.lax.dot_general(
                    dt_ref[:, j * B:(j + 1) * B], q, dn,
                    preferred_element_type=f32,
                    precision=jax.lax.Precision.HIGHEST)
            st_ref[:, ts] = acc

        @pl.when(nvalid <= float(t * B))
        def _zero(ts=ts):
            st_ref[:, ts] = jnp.zeros((8, B), f32)

    krow_ref[...] = (st_ref[4:5, :] > CONF).astype(f32)

    for b in range(NB):
        bs = slice(b * B, (b + 1) * B)
        v_row = krow_ref[0:1, bs]

        @pl.when(jnp.sum(v_row) > 0.0)
        def _process(b=b, bs=bs, v_row=v_row):
            tx1 = st_ref[0:1, bs]
            ty1 = st_ref[1:2, bs]
            tx2 = st_ref[2:3, bs]
            ty2 = st_ref[3:4, bs]
            bx1 = _row_to_col(tx1)
            by1 = _row_to_col(ty1)
            bx2 = _row_to_col(tx2)
            by2 = _row_to_col(ty2)
            m_ref[...] = _iou_mask(bx1, by1, bx2, by2, tx1, ty1, tx2, ty2)
            v_col = _row_to_col(v_row)

            ri = _iotai((B, B), 0)
            ci = _iotai((B, B), 1)
            up = (ri < ci).astype(F32)
            lo = (ri > ci).astype(F32)

            def cond(st):
                _, t, diff = st
                return (diff > 0.0) & (t < B + 2)

            def body(st):
                kc, t, _ = st
                m = m_ref[...]
                sup_r = jnp.max(m * up * kc, axis=0, keepdims=True)
                kr = v_row * (1.0 - sup_r)
                sup_c = jnp.max(m * lo * kr, axis=1, keepdims=True)
                kc2 = v_col * (1.0 - sup_c)
                return kc2, t + 1, jnp.sum(jnp.abs(kc2 - kc))

            kc_fin, _, _ = jax.lax.while_loop(
                cond, body, (v_col, jnp.int32(0), jnp.float32(1.0)))
            krow_ref[0:1, bs] = _col_to_row(kc_fin)

            @pl.when(jnp.sum(kc_fin) > 0.0)
            def _cross():
                for c in range(b + 1, NB):
                    cs = slice(c * B, (c + 1) * B)
                    ka = krow_ref[0:1, cs]

                    @pl.when(jnp.sum(ka) > 0.0)
                    def _one(c=c, cs=cs, ka=ka):
                        mt = _iou_mask(bx1, by1, bx2, by2,
                                       st_ref[0:1, cs], st_ref[1:2, cs],
                                       st_ref[2:3, cs], st_ref[3:4, cs])
                        sup = jnp.max(mt * kc_fin, axis=0, keepdims=True)
                        krow_ref[0:1, cs] = ka * (1.0 - sup)

    out_ref[...] = st_ref[...] * krow_ref[...]


@jax.jit
def kernel(x):
    p = x[0]
    xt = jnp.zeros((8, N), F32).at[:6, :N_RAW].set(p.T)
    out_t = pl.pallas_call(
        _nms_kernel,
        out_shape=jax.ShapeDtypeStruct((8, N), F32),
        scratch_shapes=[
            pltpu.VMEM((8, N), F32),
            pltpu.VMEM((8, N), F32),
            pltpu.VMEM((1, N), F32),
            pltpu.VMEM((1, N), F32),
            pltpu.VMEM((B, B), F32),
        ],
    )(xt)
    return out_t[:5, :N_RAW].T


# speedup vs baseline: 663.7791x; 1.3152x over previous
import jax
import jax.numpy as jnp
from jax.experimental import pallas as pl
from jax.experimental.pallas import tpu as pltpu

N_RAW = 5000
N = 5120
B = 512
NB = N // B
CONF = 0.25
IOU = 0.45
F32 = jnp.float32
BF16 = jnp.bfloat16


def _iotai(shape, dim):
    return jax.lax.broadcasted_iota(jnp.int32, shape, dim)


def _iota(shape, dim):
    return _iotai(shape, dim).astype(F32)


def _eye():
    return (_iotai((B, B), 0) == _iotai((B, B), 1))


def _row_to_col(v):
    return jnp.sum(jnp.where(_eye(), v, 0.0), axis=1, keepdims=True)


def _col_to_row(v):
    return jnp.sum(jnp.where(_eye(), v, 0.0), axis=0, keepdims=True)


def _iou_mask(bx1, by1, bx2, by2, a1, tx1, ty1, tx2, ty2, a2):
    ix1 = jnp.maximum(bx1, tx1)
    iy1 = jnp.maximum(by1, ty1)
    ix2 = jnp.minimum(bx2, tx2)
    iy2 = jnp.minimum(by2, ty2)
    inter = jnp.clip(ix2 - ix1, 0.0) * jnp.clip(iy2 - iy1, 0.0)
    iou = inter / (a1 + a2 - inter + 1e-9)
    return (iou > IOU).astype(F32)


def _nms_kernel(xt_ref, out_ref, dt_ref, st_ref, rkc_ref, krow_ref,
                mu_ref, ml_ref, h1_ref, h2_ref, h3_ref):
    f32 = F32
    xr = xt_ref[...]
    cxr = xr[0:1, :] * 640.0
    cyr = xr[1:2, :] * 640.0
    wr = xr[2:3, :] * 640.0
    hr = xr[3:4, :] * 640.0
    sr = xr[4:5, :] * xr[5:6, :]
    smr = jnp.where(sr > CONF, sr, -1.0)
    dtv = jnp.concatenate(
        [cxr - wr / 2, cyr - hr / 2, cxr + wr / 2, cyr + hr / 2, smr,
         jnp.zeros((3, N), f32)], axis=0)
    dt_ref[...] = dtv

    h1 = dtv.astype(BF16)
    r1 = dtv - h1.astype(f32)
    h2 = r1.astype(BF16)
    h3 = (r1 - h2.astype(f32)).astype(BF16)
    h1_ref[...] = h1
    h2_ref[...] = h2
    h3_ref[...] = h3

    tri = _iotai((B, 1), 0) > _iotai((1, B), 1)
    for t in range(NB):
        si = _row_to_col(dt_ref[4:5, t * B:(t + 1) * B])
        acc = jnp.zeros((B, 1), f32)
        for j in range(NB):
            sj = dt_ref[4:5, j * B:(j + 1) * B]
            if j < t:
                hit = sj >= si
            elif j > t:
                hit = sj > si
            else:
                hit = (sj > si) | ((sj == si) & tri)
            acc = acc + jnp.sum(hit.astype(f32), axis=1, keepdims=True)
        rkc_ref[:, t:t + 1] = acc

    dn = (((1,), (0,)), ((), ()))
    nvalid = jnp.sum((smr > CONF).astype(f32))
    for t in range(NB):
        ts = slice(t * B, (t + 1) * B)

        @pl.when(nvalid > float(t * B))
        def _gather(ts=ts, t=t):
            kkr = _iota((1, B), 1) + t * B
            acc = jnp.zeros((8, B), f32)
            for j in range(NB):
                js = slice(j * B, (j + 1) * B)
                q = (rkc_ref[:, j:j + 1] == kkr).astype(BF16)
                m1 = jax.lax.dot_general(
                    h1_ref[:, js], q, dn, preferred_element_type=f32)
                m2 = jax.lax.dot_general(
                    h2_ref[:, js], q, dn, preferred_element_type=f32)
                m3 = jax.lax.dot_general(
                    h3_ref[:, js], q, dn, preferred_element_type=f32)
                acc = acc + (m1 + (m2 + m3))
            st_ref[:, ts] = acc

        @pl.when(nvalid <= float(t * B))
        def _zero(ts=ts):
            st_ref[:, ts] = jnp.zeros((8, B), f32)

    krow_ref[...] = (st_ref[4:5, :] > CONF).astype(f32)
    ar = (jnp.clip(st_ref[2:3, :] - st_ref[0:1, :], 0.0) *
          jnp.clip(st_ref[3:4, :] - st_ref[1:2, :], 0.0))

    for b in range(NB):
        bs = slice(b * B, (b + 1) * B)
        v_row = krow_ref[0:1, bs]

        @pl.when(jnp.sum(v_row) > 0.0)
        def _process(b=b, bs=bs, v_row=v_row):
            tx1 = st_ref[0:1, bs]
            ty1 = st_ref[1:2, bs]
            tx2 = st_ref[2:3, bs]
            ty2 = st_ref[3:4, bs]
            bx1 = _row_to_col(tx1)
            by1 = _row_to_col(ty1)
            bx2 = _row_to_col(tx2)
            by2 = _row_to_col(ty2)
            a1 = (jnp.clip(bx2 - bx1, 0.0) * jnp.clip(by2 - by1, 0.0))
            m = _iou_mask(bx1, by1, bx2, by2, a1,
                          tx1, ty1, tx2, ty2, ar[0:1, bs])
            ri = _iotai((B, B), 0)
            ci = _iotai((B, B), 1)
            mu_ref[...] = m * (ri < ci).astype(F32)
            ml_ref[...] = m * (ri > ci).astype(F32)
            v_col = _row_to_col(v_row)

            def cond(st):
                _, t, diff = st
                return (diff > 0.0) & (t < B + 2)

            def body(st):
                kc, t, _ = st
                sup_r = jnp.max(mu_ref[...] * kc, axis=0, keepdims=True)
                kr = v_row * (1.0 - sup_r)
                sup_c = jnp.max(ml_ref[...] * kr, axis=1, keepdims=True)
                kc2 = v_col * (1.0 - sup_c)
                return kc2, t + 1, jnp.sum(jnp.abs(kc2 - kc))

            kc_fin, _, _ = jax.lax.while_loop(
                cond, body, (v_col, jnp.int32(0), jnp.float32(1.0)))
            krow_ref[0:1, bs] = _col_to_row(kc_fin)

            @pl.when(jnp.sum(kc_fin) > 0.0)
            def _cross():
                for c in range(b + 1, NB):
                    cs = slice(c * B, (c + 1) * B)
                    ka = krow_ref[0:1, cs]

                    @pl.when(jnp.sum(ka) > 0.0)
                    def _one(c=c, cs=cs, ka=ka):
                        mt = _iou_mask(bx1, by1, bx2, by2, a1,
                                       st_ref[0:1, cs], st_ref[1:2, cs],
                                       st_ref[2:3, cs], st_ref[3:4, cs],
                                       ar[0:1, cs])
                        sup = jnp.max(mt * kc_fin, axis=0, keepdims=True)
                        krow_ref[0:1, cs] = ka * (1.0 - sup)

    out_ref[...] = st_ref[...] * krow_ref[...]


@jax.jit
def kernel(x):
    p = x[0]
    xt = jnp.zeros((8, N), F32).at[:6, :N_RAW].set(p.T)
    out_t = pl.pallas_call(
        _nms_kernel,
        out_shape=jax.ShapeDtypeStruct((8, N), F32),
        scratch_shapes=[
            pltpu.VMEM((8, N), F32),
            pltpu.VMEM((8, N), F32),
            pltpu.VMEM((B, 16), F32),
            pltpu.VMEM((1, N), F32),
            pltpu.VMEM((B, B), F32),
            pltpu.VMEM((B, B), F32),
            pltpu.VMEM((8, N), BF16),
            pltpu.VMEM((8, N), BF16),
            pltpu.VMEM((8, N), BF16),
        ],
    )(xt)
    return out_t[:5, :N_RAW].T
